# baseline (speedup 1.0000x reference)
import jax
import jax.numpy as jnp
from jax import lax
from jax.experimental import pallas as pl
from jax.experimental.pallas import tpu as pltpu
from jax.experimental.pallas import tpu_sc as plsc

NCOLS = 26
CARD = 100
NBATCH = 16384
ROW = NCOLS * CARD
NWORKERS = 32
BPW = NBATCH // NWORKERS
LANES = 128
BLK_R = 2 * CARD
NGRP = BPW // LANES
NBLK_R = ROW // BLK_R
NBLK = NGRP * NBLK_R


NBUF = 4


def _onehot_body(xt_hbm, out_hbm, xt_v, buf0, buf1, buf2, buf3,
                 sem0, sem1, sem2, sem3):
    wid = lax.axis_index("s") * 2 + lax.axis_index("c")
    lanes = lax.iota(jnp.int32, 16)
    ones = jnp.full((16,), 1, jnp.int32)
    zeros = jnp.zeros((16,), jnp.int32)

    b0 = wid * BPW
    pltpu.sync_copy(xt_hbm.at[:, pl.ds(b0, BPW)], xt_v)

    bufs = (buf0, buf1, buf2, buf3)
    sems = (sem0, sem1, sem2, sem3)

    def zbody(r, carry):
        for s in range(LANES // 16):
            for bb in bufs:
                bb[r, pl.ds(s * 16, 16)] = zeros
        return carry
    lax.fori_loop(0, BLK_R, zbody, 0)

    def mark(t, buf, val):
        g = t // NBLK_R
        k = t % NBLK_R
        for j in range(2):
            ii = 2 * k + j
            for s in range(LANES // 16):
                xv = xt_v[ii, pl.ds(g * LANES + s * 16, 16)]
                plsc.store_scatter(
                    buf, [j * CARD + xv, s * 16 + lanes], val)

    def _dst(t):
        g = t // NBLK_R
        k = t % NBLK_R
        return out_hbm.at[pl.ds(k * BLK_R, BLK_R),
                          pl.ds(b0 + g * LANES, LANES)]

    def start(t, buf, sem):
        pltpu.async_copy(buf, _dst(t), sem)

    def wait(t, buf, sem):
        pltpu.make_async_copy(buf, _dst(t), sem).wait()

    for b in range(NBUF):
        mark(jnp.int32(b), bufs[b], ones)
        start(jnp.int32(b), bufs[b], sems[b])

    def group_body(p, carry):
        for b in range(NBUF):
            t = p * NBUF + b
            wait(t - NBUF, bufs[b], sems[b])
            mark(t - NBUF, bufs[b], zeros)
            mark(t, bufs[b], ones)
            start(t, bufs[b], sems[b])
        return carry
    lax.fori_loop(1, NBLK // NBUF, group_body, 0)

    for b in range(NBUF):
        wait(jnp.int32(NBLK - NBUF + b), bufs[b], sems[b])


def kernel(x):
    xt = x.T
    mesh = plsc.VectorSubcoreMesh(core_axis_name="c", subcore_axis_name="s")
    out_t = pl.kernel(
        _onehot_body,
        out_type=jax.ShapeDtypeStruct((ROW, NBATCH), jnp.int32),
        mesh=mesh,
        compiler_params=pltpu.CompilerParams(
            needs_layout_passes=False, use_tc_tiling_on_sc=True),
        scratch_types=[
            pltpu.VMEM((NCOLS, BPW), jnp.int32),
            pltpu.VMEM((BLK_R, LANES), jnp.int32),
            pltpu.VMEM((BLK_R, LANES), jnp.int32),
            pltpu.VMEM((BLK_R, LANES), jnp.int32),
            pltpu.VMEM((BLK_R, LANES), jnp.int32),
            pltpu.SemaphoreType.DMA,
            pltpu.SemaphoreType.DMA,
            pltpu.SemaphoreType.DMA,
            pltpu.SemaphoreType.DMA,
        ],
    )(xt)
    return out_t.T


# speedup vs baseline: 1.0046x; 1.0046x over previous
import jax
import jax.numpy as jnp
from jax import lax
from jax.experimental import pallas as pl
from jax.experimental.pallas import tpu as pltpu
from jax.experimental.pallas import tpu_sc as plsc

NCOLS = 26
CARD = 100
NBATCH = 16384
ROW = NCOLS * CARD
NWORKERS = 32
BPW = NBATCH // NWORKERS
LANES = 256
BLK_R = 2 * CARD
NGRP = BPW // LANES
NBLK_R = ROW // BLK_R
NBLK = NGRP * NBLK_R


def _onehot_body(xt_hbm, out_hbm, xt_v, buf0, buf1, sem0, sem1):
    wid = lax.axis_index("s") * 2 + lax.axis_index("c")
    lanes = lax.iota(jnp.int32, 16)
    ones = jnp.full((16,), 1, jnp.int32)
    zeros = jnp.zeros((16,), jnp.int32)

    b0 = wid * BPW
    pltpu.sync_copy(xt_hbm.at[:, pl.ds(b0, BPW)], xt_v)

    def zbody(r, carry):
        for s in range(LANES // 16):
            buf0[r, pl.ds(s * 16, 16)] = zeros
            buf1[r, pl.ds(s * 16, 16)] = zeros
        return carry
    lax.fori_loop(0, BLK_R, zbody, 0)

    bufs = (buf0, buf1)
    sems = (sem0, sem1)

    def mark(t, buf, val):
        g = t // NBLK_R
        k = t % NBLK_R
        def body(j, carry):
            ii = 2 * k + j
            for s in range(LANES // 16):
                xv = xt_v[ii, pl.ds(g * LANES + s * 16, 16)]
                plsc.store_scatter(
                    buf, [j * CARD + xv, s * 16 + lanes], val)
            return carry
        lax.fori_loop(0, 2, body, 0)

    def _dst(t):
        g = t // NBLK_R
        k = t % NBLK_R
        return out_hbm.at[pl.ds(k * BLK_R, BLK_R),
                          pl.ds(b0 + g * LANES, LANES)]

    def start(t, buf, sem):
        pltpu.async_copy(buf, _dst(t), sem)

    def wait(t, buf, sem):
        pltpu.make_async_copy(buf, _dst(t), sem).wait()

    for b in range(2):
        mark(jnp.int32(b), bufs[b], ones)
        start(jnp.int32(b), bufs[b], sems[b])

    def pair_body(p, carry):
        for b in range(2):
            t = p * 2 + b
            wait(t - 2, bufs[b], sems[b])
            mark(t - 2, bufs[b], zeros)
            mark(t, bufs[b], ones)
            start(t, bufs[b], sems[b])
        return carry
    lax.fori_loop(1, NBLK // 2, pair_body, 0)

    for b in range(2):
        wait(jnp.int32(NBLK - 2 + b), bufs[b], sems[b])


def kernel(x):
    xt = x.T
    mesh = plsc.VectorSubcoreMesh(core_axis_name="c", subcore_axis_name="s")
    out_t = pl.kernel(
        _onehot_body,
        out_type=jax.ShapeDtypeStruct((ROW, NBATCH), jnp.int32),
        mesh=mesh,
        compiler_params=pltpu.CompilerParams(
            needs_layout_passes=False, use_tc_tiling_on_sc=True),
        scratch_types=[
            pltpu.VMEM((NCOLS, BPW), jnp.int32),
            pltpu.VMEM((BLK_R, LANES), jnp.int32),
            pltpu.VMEM((BLK_R, LANES), jnp.int32),
            pltpu.SemaphoreType.DMA,
            pltpu.SemaphoreType.DMA,
        ],
    )(xt)
    return out_t.T


# speedup vs baseline: 1.0376x; 1.0329x over previous
import jax
import jax.numpy as jnp
from jax import lax
from jax.experimental import pallas as pl
from jax.experimental.pallas import tpu as pltpu
from jax.experimental.pallas import tpu_sc as plsc

NCOLS = 26
CARD = 100
NBATCH = 16384
ROW = NCOLS * CARD
NWORKERS = 32
BPW = NBATCH // NWORKERS
LANES = 128
BLK_R = 2 * CARD
NGRP = BPW // LANES
NBLK_R = ROW // BLK_R
NBLK = NGRP * NBLK_R


def _onehot_body(xt_hbm, out_hbm, xt_v, buf0, buf1, sem0, sem1):
    wid = lax.axis_index("s") * 2 + lax.axis_index("c")
    lanes = lax.iota(jnp.int32, 16)
    ones = jnp.full((16,), 1, jnp.int32)
    zeros = jnp.zeros((16,), jnp.int32)

    b0 = wid * BPW
    pltpu.sync_copy(xt_hbm.at[:, pl.ds(b0, BPW)], xt_v)

    def zbody(r, carry):
        for s in range(LANES // 16):
            buf0[r, pl.ds(s * 16, 16)] = zeros
            buf1[r, pl.ds(s * 16, 16)] = zeros
        return carry
    lax.fori_loop(0, BLK_R, zbody, 0)

    bufs = (buf0, buf1)
    sems = (sem0, sem1)

    def mark(t, buf, val):
        g = t // NBLK_R
        k = t % NBLK_R
        def body(j, carry):
            ii = 2 * k + j
            for s in range(LANES // 16):
                xv = xt_v[ii, pl.ds(g * LANES + s * 16, 16)]
                plsc.store_scatter(
                    buf, [j * CARD + xv, s * 16 + lanes], val)
            return carry
        lax.fori_loop(0, 2, body, 0)

    def _dst(t):
        g = t // NBLK_R
        k = t % NBLK_R
        return out_hbm.at[pl.ds(k * BLK_R, BLK_R),
                          pl.ds(b0 + g * LANES, LANES)]

    def start(t, buf, sem):
        pltpu.async_copy(buf, _dst(t), sem)

    def wait(t, buf, sem):
        pltpu.make_async_copy(buf, _dst(t), sem).wait()

    for b in range(2):
        mark(jnp.int32(b), bufs[b], ones)
        start(jnp.int32(b), bufs[b], sems[b])

    def pair_body(p, carry):
        for b in range(2):
            t = p * 2 + b
            wait(t - 2, bufs[b], sems[b])
            mark(t - 2, bufs[b], zeros)
            mark(t, bufs[b], ones)
            start(t, bufs[b], sems[b])
        return carry
    lax.fori_loop(1, NBLK // 2, pair_body, 0)

    for b in range(2):
        wait(jnp.int32(NBLK - 2 + b), bufs[b], sems[b])


def kernel(x):
    xt = x.T
    mesh = plsc.VectorSubcoreMesh(core_axis_name="c", subcore_axis_name="s")
    out_t = pl.kernel(
        _onehot_body,
        out_type=jax.ShapeDtypeStruct((ROW, NBATCH), jnp.int32),
        mesh=mesh,
        compiler_params=pltpu.CompilerParams(
            needs_layout_passes=False, use_tc_tiling_on_sc=True,
            disable_bounds_checks=True),
        scratch_types=[
            pltpu.VMEM((NCOLS, BPW), jnp.int32),
            pltpu.VMEM((BLK_R, LANES), jnp.int32),
            pltpu.VMEM((BLK_R, LANES), jnp.int32),
            pltpu.SemaphoreType.DMA,
            pltpu.SemaphoreType.DMA,
        ],
    )(xt)
    return out_t.T


# speedup vs baseline: 1.0649x; 1.0263x over previous
import jax
import jax.numpy as jnp
from jax import lax
from jax.experimental import pallas as pl
from jax.experimental.pallas import tpu as pltpu
from jax.experimental.pallas import tpu_sc as plsc

NCOLS = 26
CARD = 100
NBATCH = 16384
ROW = NCOLS * CARD
NWORKERS = 32
BPW = NBATCH // NWORKERS
LANES = 128
BLK_R = 2 * CARD
NGRP = BPW // LANES
NBLK_R = ROW // BLK_R
NBLK = NGRP * NBLK_R


def _onehot_body(xt_hbm, out_hbm, xt_v, buf0, buf1, sem0, sem1):
    wid = lax.axis_index("s") * 2 + lax.axis_index("c")
    lanes = lax.iota(jnp.int32, 16)
    ones = jnp.full((16,), 1, jnp.int32)
    zeros = jnp.zeros((16,), jnp.int32)

    b0 = wid * BPW
    xcopy = pltpu.async_copy(xt_hbm.at[:, pl.ds(b0, BPW)], xt_v, sem1)

    def zfill(buf):
        def zbody(r, carry):
            for s in range(LANES // 16):
                buf[r, pl.ds(s * 16, 16)] = zeros
            return carry
        lax.fori_loop(0, BLK_R, zbody, 0)

    zfill(buf0)
    xcopy.wait()

    bufs = (buf0, buf1)
    sems = (sem0, sem1)

    def mark(t, buf, val):
        g = t // NBLK_R
        k = t % NBLK_R
        def body(j, carry):
            ii = 2 * k + j
            for s in range(LANES // 16):
                xv = xt_v[ii, pl.ds(g * LANES + s * 16, 16)]
                plsc.store_scatter(
                    buf, [j * CARD + xv, s * 16 + lanes], val)
            return carry
        lax.fori_loop(0, 2, body, 0)

    def _dst(t):
        g = t // NBLK_R
        k = t % NBLK_R
        return out_hbm.at[pl.ds(k * BLK_R, BLK_R),
                          pl.ds(b0 + g * LANES, LANES)]

    def start(t, buf, sem):
        pltpu.async_copy(buf, _dst(t), sem)

    def wait(t, buf, sem):
        pltpu.make_async_copy(buf, _dst(t), sem).wait()

    mark(jnp.int32(0), buf0, ones)
    start(jnp.int32(0), buf0, sem0)
    zfill(buf1)
    mark(jnp.int32(1), buf1, ones)
    start(jnp.int32(1), buf1, sem1)

    def pair_body(p, carry):
        for b in range(2):
            t = p * 2 + b
            wait(t - 2, bufs[b], sems[b])
            mark(t - 2, bufs[b], zeros)
            mark(t, bufs[b], ones)
            start(t, bufs[b], sems[b])
        return carry
    lax.fori_loop(1, NBLK // 2, pair_body, 0)

    for b in range(2):
        wait(jnp.int32(NBLK - 2 + b), bufs[b], sems[b])


def kernel(x):
    xt = x.T
    mesh = plsc.VectorSubcoreMesh(core_axis_name="c", subcore_axis_name="s")
    out_t = pl.kernel(
        _onehot_body,
        out_type=jax.ShapeDtypeStruct((ROW, NBATCH), jnp.int32),
        mesh=mesh,
        compiler_params=pltpu.CompilerParams(
            needs_layout_passes=False, use_tc_tiling_on_sc=True,
            disable_bounds_checks=True, skip_device_barrier=True),
        scratch_types=[
            pltpu.VMEM((NCOLS, BPW), jnp.int32),
            pltpu.VMEM((BLK_R, LANES), jnp.int32),
            pltpu.VMEM((BLK_R, LANES), jnp.int32),
            pltpu.SemaphoreType.DMA,
            pltpu.SemaphoreType.DMA,
        ],
    )(xt)
    return out_t.T
